# baseline (speedup 1.0000x reference)
import functools
import math

import jax
import jax.numpy as jnp
from jax import lax
from jax.experimental import pallas as pl
from jax.experimental.pallas import tpu as pltpu


def _se_packed_kernel(x_ref, w1a_ref, w2p_ref, o_ref, *, hw, g, cr):
    xb = x_ref[0]
    cols = [jnp.sum(xb[:, k * hw:(k + 1) * hw], axis=1, keepdims=True)
            for k in range(g)]
    pooled = jnp.concatenate(cols, axis=1) * (1.0 / hw)
    out1 = lax.dot_general(pooled, w1a_ref[...], (((0,), (0,)), ((), ())),
                           preferred_element_type=jnp.float32)
    h = out1[0:1, 0:cr]
    for k in range(1, g):
        h = h + out1[k:k + 1, cr * k:cr * k + cr]
    h_row = jnp.maximum(h, 0.0)
    prod = w2p_ref[...] * jnp.concatenate([h_row] * g, axis=1)
    sh = cr // 2
    while sh >= 1:
        prod = prod + jnp.roll(prod, -sh, axis=1)
        sh //= 2
    swide = jax.nn.sigmoid(prod)
    seg = lax.broadcasted_iota(jnp.int32, (1, g * hw), 1) // hw
    sexp = jnp.broadcast_to(swide[:, (g - 1) * cr:(g - 1) * cr + 1], xb.shape)
    for k in range(g - 2, -1, -1):
        sexp = jnp.where(seg == k, swide[:, cr * k:cr * k + 1], sexp)
    o_ref[0] = xb * sexp


def _se_simple_kernel(x_ref, w1_ref, w2_ref, o_ref, *, inv_hw):
    xb = x_ref[0]
    pooled = jnp.sum(xb, axis=1, keepdims=True) * inv_hw
    h = jnp.maximum(
        jax.lax.dot_general(w1_ref[...], pooled, (((1,), (0,)), ((), ())),
                            preferred_element_type=jnp.float32), 0.0)
    s = jax.nn.sigmoid(
        jax.lax.dot_general(w2_ref[...], h, (((1,), (0,)), ((), ())),
                            preferred_element_type=jnp.float32))
    o_ref[0] = xb * s


def kernel(x, w1, w2):
    B, C, H, W = x.shape
    HW = H * W
    Cr = w1.shape[0]
    G = 128 // math.gcd(HW, 128)

    packed_ok = (
        C % G == 0 and (C // G) % 8 == 0
        and 128 % Cr == 0 and 128 // Cr == G and Cr & (Cr - 1) == 0
    )
    if packed_ok:
        xp = x.reshape(B, C // G, G * HW)
        w2p = w2.reshape((C * Cr) // 128, 128)
        w1a = jnp.transpose(w1.reshape(Cr, C // G, G), (1, 2, 0))
        w1a = w1a.reshape(C // G, G * Cr)
        body = functools.partial(_se_packed_kernel, hw=HW, g=G, cr=Cr)
        out = pl.pallas_call(
            body,
            out_shape=jax.ShapeDtypeStruct(xp.shape, x.dtype),
            grid=(B,),
            in_specs=[
                pl.BlockSpec((1, C // G, G * HW), lambda b: (b, 0, 0)),
                pl.BlockSpec((C // G, G * Cr), lambda b: (0, 0)),
                pl.BlockSpec(((C * Cr) // 128, 128), lambda b: (0, 0)),
            ],
            out_specs=pl.BlockSpec((1, C // G, G * HW), lambda b: (b, 0, 0)),
            compiler_params=pltpu.CompilerParams(
                dimension_semantics=("parallel",),
            ),
        )(xp, w1a, w2p)
        return out.reshape(B, C, H, W)

    x3 = x.reshape(B, C, HW)
    body = functools.partial(_se_simple_kernel, inv_hw=1.0 / float(HW))
    out3 = pl.pallas_call(
        body,
        out_shape=jax.ShapeDtypeStruct((B, C, HW), x.dtype),
        grid=(B,),
        in_specs=[
            pl.BlockSpec((1, C, HW), lambda b: (b, 0, 0)),
            pl.BlockSpec((Cr, C), lambda b: (0, 0)),
            pl.BlockSpec((C, Cr), lambda b: (0, 0)),
        ],
        out_specs=pl.BlockSpec((1, C, HW), lambda b: (b, 0, 0)),
        compiler_params=pltpu.CompilerParams(
            dimension_semantics=("parallel",),
        ),
    )(x3, w1, w2)
    return out3.reshape(B, C, H, W)


# speedup vs baseline: 1.0043x; 1.0043x over previous
import functools
import math

import jax
import jax.numpy as jnp
from jax import lax
from jax.experimental import pallas as pl
from jax.experimental.pallas import tpu as pltpu


def _se_pipeline_kernel(x_hbm, w1a_ref, w2p_ref, q_ref, o_hbm,
                        xbuf, obuf, in_sem, out_sem, *, n, hw, g, cr):
    def dma_in(slot, step):
        pltpu.make_async_copy(x_hbm.at[step], xbuf.at[slot],
                              in_sem.at[slot]).start()

    def wait_in(slot):
        pltpu.make_async_copy(xbuf.at[slot], xbuf.at[slot],
                              in_sem.at[slot]).wait()

    def dma_out(slot, step):
        pltpu.make_async_copy(obuf.at[slot], o_hbm.at[step],
                              out_sem.at[slot]).start()

    def wait_out(slot):
        pltpu.make_async_copy(obuf.at[slot], obuf.at[slot],
                              out_sem.at[slot]).wait()

    dma_in(0, 0)

    def body(step, _):
        cur = lax.rem(step, 2)
        nxt = lax.rem(step + 1, 2)

        @pl.when(step + 1 < n)
        def _():
            dma_in(nxt, step + 1)

        wait_in(cur)

        @pl.when(step >= 2)
        def _():
            wait_out(cur)

        xb = xbuf[cur]
        cols = [jnp.sum(xb[:, k * hw:(k + 1) * hw], axis=1, keepdims=True)
                for k in range(g)]
        pooled = jnp.concatenate(cols, axis=1) * (1.0 / hw)
        out1 = lax.dot_general(pooled, w1a_ref[...],
                               (((0,), (0,)), ((), ())),
                               preferred_element_type=jnp.float32)
        h = out1[0:1, 0:cr]
        for k in range(1, g):
            h = h + out1[k:k + 1, cr * k:cr * k + cr]
        h_row = jnp.maximum(h, 0.0)
        prod = w2p_ref[...] * jnp.concatenate([h_row] * g, axis=1)
        sh = cr // 2
        while sh >= 1:
            prod = prod + jnp.roll(prod, -sh, axis=1)
            sh //= 2
        swide = jax.nn.sigmoid(prod)
        sexp = lax.dot_general(swide.astype(jnp.bfloat16), q_ref[...],
                               (((1,), (0,)), ((), ())),
                               preferred_element_type=jnp.float32)
        obuf[cur] = xb * sexp
        dma_out(cur, step)
        return ()

    lax.fori_loop(0, n, body, ())
    if n >= 2:
        wait_out(lax.rem(n - 2, 2))
    wait_out(lax.rem(n - 1, 2))


def _se_simple_kernel(x_ref, w1_ref, w2_ref, o_ref, *, inv_hw):
    xb = x_ref[0]
    pooled = jnp.sum(xb, axis=1, keepdims=True) * inv_hw
    h = jnp.maximum(
        lax.dot_general(w1_ref[...], pooled, (((1,), (0,)), ((), ())),
                        preferred_element_type=jnp.float32), 0.0)
    s = jax.nn.sigmoid(
        lax.dot_general(w2_ref[...], h, (((1,), (0,)), ((), ())),
                        preferred_element_type=jnp.float32))
    o_ref[0] = xb * s


def kernel(x, w1, w2):
    B, C, H, W = x.shape
    HW = H * W
    Cr = w1.shape[0]
    G = 128 // math.gcd(HW, 128)

    packed_ok = (
        B >= 2 and C % G == 0 and (C // G) % 8 == 0
        and 128 % Cr == 0 and 128 // Cr == G and Cr & (Cr - 1) == 0
    )
    if packed_ok:
        R, L = C // G, G * HW
        xp = x.reshape(B, R, L)
        w2p = w2.reshape((C * Cr) // 128, 128)
        w1a = jnp.transpose(w1.reshape(Cr, R, G), (1, 2, 0)).reshape(R, G * Cr)
        q = (lax.broadcasted_iota(jnp.int32, (128, L), 0) ==
             Cr * (lax.broadcasted_iota(jnp.int32, (128, L), 1) // HW)
             ).astype(jnp.bfloat16)
        body = functools.partial(_se_pipeline_kernel, n=B, hw=HW, g=G, cr=Cr)
        out = pl.pallas_call(
            body,
            out_shape=jax.ShapeDtypeStruct((B, R, L), x.dtype),
            in_specs=[
                pl.BlockSpec(memory_space=pl.ANY),
                pl.BlockSpec(memory_space=pltpu.MemorySpace.VMEM),
                pl.BlockSpec(memory_space=pltpu.MemorySpace.VMEM),
                pl.BlockSpec(memory_space=pltpu.MemorySpace.VMEM),
            ],
            out_specs=pl.BlockSpec(memory_space=pl.ANY),
            scratch_shapes=[
                pltpu.VMEM((2, R, L), x.dtype),
                pltpu.VMEM((2, R, L), x.dtype),
                pltpu.SemaphoreType.DMA((2,)),
                pltpu.SemaphoreType.DMA((2,)),
            ],
        )(xp, w1a, w2p, q)
        return out.reshape(B, C, H, W)

    x3 = x.reshape(B, C, HW)
    body = functools.partial(_se_simple_kernel, inv_hw=1.0 / float(HW))
    out3 = pl.pallas_call(
        body,
        out_shape=jax.ShapeDtypeStruct((B, C, HW), x.dtype),
        grid=(B,),
        in_specs=[
            pl.BlockSpec((1, C, HW), lambda b: (b, 0, 0)),
            pl.BlockSpec((Cr, C), lambda b: (0, 0)),
            pl.BlockSpec((C, Cr), lambda b: (0, 0)),
        ],
        out_specs=pl.BlockSpec((1, C, HW), lambda b: (b, 0, 0)),
        compiler_params=pltpu.CompilerParams(
            dimension_semantics=("parallel",),
        ),
    )(x3, w1, w2)
    return out3.reshape(B, C, H, W)


# speedup vs baseline: 3.1683x; 3.1547x over previous
import functools

import jax
import jax.numpy as jnp
from jax import lax
from jax.experimental import pallas as pl
from jax.experimental.pallas import tpu as pltpu


def _se_pipeline_kernel(x_hbm, w1_ref, w2_ref, o_hbm,
                        xbuf, obuf, in_sem, out_sem, *, nb, inv_hw):
    base = pl.program_id(0) * nb

    def dma_in(slot, step):
        pltpu.make_async_copy(x_hbm.at[base + step], xbuf.at[slot],
                              in_sem.at[slot]).start()

    def wait_in(slot):
        pltpu.make_async_copy(xbuf.at[slot], xbuf.at[slot],
                              in_sem.at[slot]).wait()

    def dma_out(slot, step):
        pltpu.make_async_copy(obuf.at[slot], o_hbm.at[base + step],
                              out_sem.at[slot]).start()

    def wait_out(slot):
        pltpu.make_async_copy(obuf.at[slot], obuf.at[slot],
                              out_sem.at[slot]).wait()

    dma_in(0, 0)

    def body(step, _):
        cur = lax.rem(step, 2)
        nxt = lax.rem(step + 1, 2)

        @pl.when(step + 1 < nb)
        def _():
            dma_in(nxt, step + 1)

        wait_in(cur)

        @pl.when(step >= 2)
        def _():
            wait_out(cur)

        xb = xbuf[cur]
        pooled = jnp.sum(xb, axis=1, keepdims=True) * inv_hw
        h = jnp.maximum(
            lax.dot_general(w1_ref[...], pooled, (((1,), (0,)), ((), ())),
                            preferred_element_type=jnp.float32), 0.0)
        s = jax.nn.sigmoid(
            lax.dot_general(w2_ref[...], h, (((1,), (0,)), ((), ())),
                            preferred_element_type=jnp.float32))
        obuf[cur] = xb * s
        dma_out(cur, step)
        return ()

    lax.fori_loop(0, nb, body, ())
    if nb >= 2:
        wait_out(lax.rem(nb - 2, 2))
    wait_out(lax.rem(nb - 1, 2))


def kernel(x, w1, w2):
    B, C, H, W = x.shape
    HW = H * W
    Cr = w1.shape[0]

    x3 = x.reshape(B, C, HW)
    n_cores = 2 if B % 2 == 0 else 1
    nb = B // n_cores

    body = functools.partial(_se_pipeline_kernel, nb=nb,
                             inv_hw=1.0 / float(HW))
    out3 = pl.pallas_call(
        body,
        out_shape=jax.ShapeDtypeStruct((B, C, HW), x.dtype),
        grid=(n_cores,),
        in_specs=[
            pl.BlockSpec(memory_space=pl.ANY),
            pl.BlockSpec((Cr, C), lambda i: (0, 0)),
            pl.BlockSpec((C, Cr), lambda i: (0, 0)),
        ],
        out_specs=pl.BlockSpec(memory_space=pl.ANY),
        scratch_shapes=[
            pltpu.VMEM((2, C, HW), x.dtype),
            pltpu.VMEM((2, C, HW), x.dtype),
            pltpu.SemaphoreType.DMA((2,)),
            pltpu.SemaphoreType.DMA((2,)),
        ],
        compiler_params=pltpu.CompilerParams(
            dimension_semantics=("parallel",),
        ),
    )(x3, w1, w2)
    return out3.reshape(B, C, H, W)
